# baseline (speedup 1.0000x reference)
import functools

import jax
import jax.numpy as jnp
from jax import lax
from jax.experimental import pallas as pl
from jax.experimental.pallas import tpu as pltpu
from jax.experimental.pallas import tpu_sc as plsc

_NC = 2
_NS = 16
_NW = _NC * _NS
_LANES = 16
_CHUNK = 128


def _make_gather(B, F, V, D, blk):
    NB = B // blk
    BF = B * F
    rows_w = BF // _NW
    n_ch = rows_w // _CHUNK
    mesh = plsc.VectorSubcoreMesh(core_axis_name="c", subcore_axis_name="s")

    nbuf = 6

    @functools.partial(
        pl.kernel,
        out_type=jax.ShapeDtypeStruct((NB, F, blk, D), jnp.float32),
        mesh=mesh,
        scratch_types=[
            pltpu.VMEM((rows_w,), jnp.int32),
        ] + [pltpu.VMEM((_CHUNK,), jnp.int32) for _ in range(nbuf)]
          + [pltpu.VMEM((_CHUNK, D), jnp.float32) for _ in range(nbuf)]
          + [pltpu.SemaphoreType.DMA for _ in range(2 * nbuf)],
    )
    def gather_k(xcat_hbm, table_hbm, out_hbm, raw_v, *ring):
        idxs = ring[:nbuf]
        bufs = ring[nbuf:2 * nbuf]
        gsems = ring[2 * nbuf:3 * nbuf]
        wsems = ring[3 * nbuf:4 * nbuf]
        wid = lax.axis_index("s") * _NC + lax.axis_index("c")
        base = wid * rows_w
        pltpu.sync_copy(xcat_hbm.at[pl.ds(base, rows_w)], raw_v)

        def out_slice(c):
            p0 = base + c * _CHUNK
            f0 = p0 // B
            b0 = p0 - f0 * B
            i0 = b0 // blk
            return out_hbm.at[i0, f0, pl.ds(b0 - i0 * blk, _CHUNK)]

        def start_gather(c):
            s = c % nbuf
            off = c * _CHUNK
            voff = ((base + off) // B) * V
            for j in range(_CHUNK // _LANES):
                o = off + j * _LANES
                idxs[s][pl.ds(j * _LANES, _LANES)] = raw_v[pl.ds(o, _LANES)] + voff
            pltpu.async_copy(table_hbm.at[idxs[s]], bufs[s], gsems[s])

        depth = nbuf // 2
        for c in range(depth):
            start_gather(c)
        for c in range(n_ch):
            s = c % nbuf
            pltpu.make_async_copy(table_hbm.at[idxs[s]], bufs[s], gsems[s]).wait()
            pltpu.async_copy(bufs[s], out_slice(c), wsems[s])
            if c >= depth:
                s2 = (c - depth) % nbuf
                pltpu.make_async_copy(bufs[s2], out_slice(c - depth), wsems[s2]).wait()
            if c + depth < n_ch:
                start_gather(c + depth)
        for c in range(max(n_ch - depth, 0), n_ch):
            s = c % nbuf
            pltpu.make_async_copy(bufs[s], out_slice(c), wsems[s]).wait()

    return gather_k


def _make_mlp(B, F, D, H1, H2, OUT, blk):
    def body(x_ref, w1_ref, b1_ref, w2_ref, b2_ref, w3_ref, b3_ref, o_ref):
        xb = jnp.concatenate([x_ref[0, f] for f in range(F)],
                             axis=1).astype(jnp.bfloat16)
        h = lax.dot_general(xb, w1_ref[...], (((1,), (0,)), ((), ())),
                            preferred_element_type=jnp.float32)
        h = jnp.maximum(h + b1_ref[...], 0.0).astype(jnp.bfloat16)
        h = lax.dot_general(h, w2_ref[...], (((1,), (0,)), ((), ())),
                            preferred_element_type=jnp.float32)
        h = jnp.maximum(h + b2_ref[...], 0.0)
        o = lax.dot_general(h, w3_ref[...], (((1,), (0,)), ((), ())),
                            preferred_element_type=jnp.float32)
        o_ref[...] = jax.nn.sigmoid(o + b3_ref[...])

    return pl.pallas_call(
        body,
        grid=(B // blk,),
        in_specs=[
            pl.BlockSpec((1, F, blk, D), lambda i: (i, 0, 0, 0)),
            pl.BlockSpec((F * D, H1), lambda i: (0, 0)),
            pl.BlockSpec((1, H1), lambda i: (0, 0)),
            pl.BlockSpec((H1, H2), lambda i: (0, 0)),
            pl.BlockSpec((1, H2), lambda i: (0, 0)),
            pl.BlockSpec((H2, OUT), lambda i: (0, 0)),
            pl.BlockSpec((1, OUT), lambda i: (0, 0)),
        ],
        out_specs=pl.BlockSpec((blk, OUT), lambda i: (i, 0)),
        out_shape=jax.ShapeDtypeStruct((B, OUT), jnp.float32),
    )


_NSPLIT = 4


def kernel(x_categorical, tables, W1, b1, W2, b2, W3, b3):
    B, F = x_categorical.shape
    _, V, D = tables.shape
    H1 = W1.shape[1]
    H2 = W2.shape[1]
    OUT = W3.shape[1]

    Bs = B // _NSPLIT
    tab_flat = tables.reshape(F * V, D)
    blk = 1024
    gather = _make_gather(Bs, F, V, D, blk)
    mlp = _make_mlp(Bs, F, D, H1, H2, OUT, blk)
    w1b = W1.astype(jnp.bfloat16)
    w2b = W2.astype(jnp.bfloat16)
    b1r, b2r, b3r = b1.reshape(1, H1), b2.reshape(1, H2), b3.reshape(1, OUT)

    outs = []
    for h in range(_NSPLIT):
        xcat_fm = x_categorical[h * Bs:(h + 1) * Bs].T.reshape(F * Bs)
        gathered = gather(xcat_fm, tab_flat)
        outs.append(mlp(gathered, w1b, b1r, w2b, b2r, W3, b3r))
    return jnp.concatenate(outs, axis=0)


# speedup vs baseline: 1.0102x; 1.0102x over previous
import functools

import jax
import jax.numpy as jnp
from jax import lax
from jax.experimental import pallas as pl
from jax.experimental.pallas import tpu as pltpu
from jax.experimental.pallas import tpu_sc as plsc

_NC = 2
_NS = 16
_NW = _NC * _NS
_LANES = 16
_CHUNK = 128


def _make_gather(B, F, V, D, blk):
    NB = B // blk
    BF = B * F
    rows_w = BF // _NW
    n_ch = rows_w // _CHUNK
    mesh = plsc.VectorSubcoreMesh(core_axis_name="c", subcore_axis_name="s")

    nbuf = 4

    @functools.partial(
        pl.kernel,
        out_type=jax.ShapeDtypeStruct((NB, F, blk, D), jnp.float32),
        mesh=mesh,
        scratch_types=[
            pltpu.VMEM((rows_w,), jnp.int32),
        ] + [pltpu.VMEM((_CHUNK,), jnp.int32) for _ in range(nbuf)]
          + [pltpu.VMEM((_CHUNK, D), jnp.float32) for _ in range(nbuf)]
          + [pltpu.SemaphoreType.DMA for _ in range(2 * nbuf)],
    )
    def gather_k(xcat_hbm, table_hbm, out_hbm, raw_v, *ring):
        idxs = ring[:nbuf]
        bufs = ring[nbuf:2 * nbuf]
        gsems = ring[2 * nbuf:3 * nbuf]
        wsems = ring[3 * nbuf:4 * nbuf]
        wid = lax.axis_index("s") * _NC + lax.axis_index("c")
        base = wid * rows_w
        pltpu.sync_copy(xcat_hbm.at[pl.ds(base, rows_w)], raw_v)

        def out_slice(c):
            p0 = base + c * _CHUNK
            f0 = p0 // B
            b0 = p0 - f0 * B
            i0 = b0 // blk
            return out_hbm.at[i0, f0, pl.ds(b0 - i0 * blk, _CHUNK)]

        def start_gather(c):
            s = c % nbuf
            off = c * _CHUNK
            voff = ((base + off) // B) * V
            for j in range(_CHUNK // _LANES):
                o = off + j * _LANES
                idxs[s][pl.ds(j * _LANES, _LANES)] = raw_v[pl.ds(o, _LANES)] + voff
            pltpu.async_copy(table_hbm.at[idxs[s]], bufs[s], gsems[s])

        depth = nbuf // 2
        for c in range(depth):
            start_gather(c)
        for c in range(n_ch):
            s = c % nbuf
            pltpu.make_async_copy(table_hbm.at[idxs[s]], bufs[s], gsems[s]).wait()
            pltpu.async_copy(bufs[s], out_slice(c), wsems[s])
            if c >= depth:
                s2 = (c - depth) % nbuf
                pltpu.make_async_copy(bufs[s2], out_slice(c - depth), wsems[s2]).wait()
            if c + depth < n_ch:
                start_gather(c + depth)
        for c in range(max(n_ch - depth, 0), n_ch):
            s = c % nbuf
            pltpu.make_async_copy(bufs[s], out_slice(c), wsems[s]).wait()

    return gather_k


def _make_mlp(B, F, D, H1, H2, OUT, blk):
    def body(x_ref, w1_ref, b1_ref, w2_ref, b2_ref, w3_ref, b3_ref, o_ref):
        xb = jnp.concatenate([x_ref[0, f] for f in range(F)],
                             axis=1).astype(jnp.bfloat16)
        h = lax.dot_general(xb, w1_ref[...], (((1,), (0,)), ((), ())),
                            preferred_element_type=jnp.float32)
        h = jnp.maximum(h + b1_ref[...], 0.0).astype(jnp.bfloat16)
        h = lax.dot_general(h, w2_ref[...], (((1,), (0,)), ((), ())),
                            preferred_element_type=jnp.float32)
        h = jnp.maximum(h + b2_ref[...], 0.0)
        o = lax.dot_general(h, w3_ref[...], (((1,), (0,)), ((), ())),
                            preferred_element_type=jnp.float32)
        o_ref[...] = jax.nn.sigmoid(o + b3_ref[...])

    return pl.pallas_call(
        body,
        grid=(B // blk,),
        in_specs=[
            pl.BlockSpec((1, F, blk, D), lambda i: (i, 0, 0, 0)),
            pl.BlockSpec((F * D, H1), lambda i: (0, 0)),
            pl.BlockSpec((1, H1), lambda i: (0, 0)),
            pl.BlockSpec((H1, H2), lambda i: (0, 0)),
            pl.BlockSpec((1, H2), lambda i: (0, 0)),
            pl.BlockSpec((H2, OUT), lambda i: (0, 0)),
            pl.BlockSpec((1, OUT), lambda i: (0, 0)),
        ],
        out_specs=pl.BlockSpec((blk, OUT), lambda i: (i, 0)),
        out_shape=jax.ShapeDtypeStruct((B, OUT), jnp.float32),
    )


_NSPLIT = 4


def kernel(x_categorical, tables, W1, b1, W2, b2, W3, b3):
    B, F = x_categorical.shape
    _, V, D = tables.shape
    H1 = W1.shape[1]
    H2 = W2.shape[1]
    OUT = W3.shape[1]

    Bs = B // _NSPLIT
    tab_flat = tables.reshape(F * V, D)
    blk = 1024
    gather = _make_gather(Bs, F, V, D, blk)
    mlp = _make_mlp(Bs, F, D, H1, H2, OUT, blk)
    w1b = W1.astype(jnp.bfloat16)
    w2b = W2.astype(jnp.bfloat16)
    b1r, b2r, b3r = b1.reshape(1, H1), b2.reshape(1, H2), b3.reshape(1, OUT)

    outs = []
    for h in range(_NSPLIT):
        xcat_fm = x_categorical[h * Bs:(h + 1) * Bs].T.reshape(F * Bs)
        gathered = gather(xcat_fm, tab_flat)
        outs.append(mlp(gathered, w1b, b1r, w2b, b2r, W3, b3r))
    return jnp.concatenate(outs, axis=0)
